# baseline (speedup 1.0000x reference)
import functools

import jax
import jax.numpy as jnp
from jax.experimental import pallas as pl
from jax.experimental.pallas import tpu as pltpu

LARGEST_RETRIEVAL = 64
NO_HARD_NEG = 10
NO_PSEUDO_POS = 2

_B = 1024
_D = 128
_N = 100000
_BT = 256
_W = 4096
_NPAD = 102400
_BIN = 32
_NBINS_STEP = _W // _BIN
_MPAD = _NPAD // _BIN
_NCAND = LARGEST_RETRIEVAL * _BIN


def _l2n(x):
    n = jnp.linalg.norm(x, axis=-1, keepdims=True)
    return x / jnp.maximum(n, 1e-12)


def _score_binmax_kernel(q_ref, sq_ref, t_ref, st_ref, s_ref, m_ref):
    s = jax.lax.dot_general(
        q_ref[...], t_ref[...], (((1,), (1,)), ((), ())),
        preferred_element_type=jnp.float32,
    )
    neg_d2 = -((sq_ref[...] + st_ref[0]) - 2.0 * s)

    j = pl.program_id(1)
    gid = j * _W + jax.lax.broadcasted_iota(jnp.int32, (_BT, _W), 1)
    v = jnp.where(gid < _N, neg_d2, -1e30)
    s_ref[...] = v
    m_ref[...] = jnp.max(v.reshape(_BT, _NBINS_STEP, _BIN), axis=2)


def _topk_iter(v, k):
    rows, L = v.shape
    lane = jax.lax.broadcasted_iota(jnp.int32, (rows, L), 1)
    kcol = jax.lax.broadcasted_iota(jnp.int32, (rows, k), 1)

    def body(i, carry):
        m, pos = carry
        am = jnp.argmax(m, axis=1).astype(jnp.int32)
        pos = jnp.where(kcol == i, am[:, None], pos)
        m = jnp.where(lane == am[:, None], -jnp.inf, m)
        return m, pos

    _, pos = jax.lax.fori_loop(
        0, k, body, (v, jnp.zeros((rows, k), jnp.int32)))
    return pos


def _binpick_kernel(m_ref, ids_ref):
    pos = _topk_iter(m_ref[...], LARGEST_RETRIEVAL)
    kcol = jax.lax.broadcasted_iota(jnp.int32, (_BT, LARGEST_RETRIEVAL), 1)

    def body(i, carry):
        ids, out = carry
        mn = jnp.min(ids, axis=1, keepdims=True)
        out = jnp.where(kcol == i, mn, out)
        ids = jnp.where(ids == mn, _MPAD, ids)
        return ids, out

    _, ids_sorted = jax.lax.fori_loop(
        0, LARGEST_RETRIEVAL, body,
        (pos, jnp.zeros((_BT, LARGEST_RETRIEVAL), jnp.int32)))
    ids_ref[...] = ids_sorted


def _candpick_kernel(cv_ref, pos_ref):
    pos_ref[...] = _topk_iter(cv_ref[...], LARGEST_RETRIEVAL)


def _sel_ranks(mask, quota):
    rank = jnp.cumsum(mask.astype(jnp.int32), axis=1) - 1
    valid = mask & (rank < quota)
    oh = valid[:, :, None] & (rank[:, :, None] == jnp.arange(quota)[None, None, :])
    r_s = jnp.sum(jnp.where(oh, jnp.arange(mask.shape[1])[None, :, None], 0),
                  axis=1)
    filled = jnp.any(oh, axis=1)
    return r_s, filled


@jax.jit
def kernel(query_feats, query_labels, train_feats, train_labels):
    q = _l2n(query_feats)
    t = _l2n(train_feats)
    sq = jnp.sum(q * q, axis=1)
    st = jnp.sum(t * t, axis=1)

    t_pad = jnp.pad(t, ((0, _NPAD - _N), (0, 0)))
    st_pad = jnp.pad(st, (0, _NPAD - _N)).reshape(_NPAD // _W, 1, _W)
    sq2 = sq.reshape(_B, 1)

    scores, binmax = pl.pallas_call(
        _score_binmax_kernel,
        grid=(_B // _BT, _NPAD // _W),
        in_specs=[
            pl.BlockSpec((_BT, _D), lambda i, j: (i, 0)),
            pl.BlockSpec((_BT, 1), lambda i, j: (i, 0)),
            pl.BlockSpec((_W, _D), lambda i, j: (j, 0)),
            pl.BlockSpec((1, 1, _W), lambda i, j: (j, 0, 0)),
        ],
        out_specs=[
            pl.BlockSpec((_BT, _W), lambda i, j: (i, j)),
            pl.BlockSpec((_BT, _NBINS_STEP), lambda i, j: (i, j)),
        ],
        out_shape=[
            jax.ShapeDtypeStruct((_B, _NPAD), jnp.float32),
            jax.ShapeDtypeStruct((_B, _MPAD), jnp.float32),
        ],
    )(q, sq2, t_pad, st_pad)

    bin_ids = pl.pallas_call(
        _binpick_kernel,
        grid=(_B // _BT,),
        in_specs=[pl.BlockSpec((_BT, _MPAD), lambda i: (i, 0))],
        out_specs=pl.BlockSpec((_BT, LARGEST_RETRIEVAL), lambda i: (i, 0)),
        out_shape=jax.ShapeDtypeStruct((_B, LARGEST_RETRIEVAL), jnp.int32),
    )(binmax)

    col_base = bin_ids * _BIN
    cand_idx = (col_base[:, :, None] + jnp.arange(_BIN)[None, None, :]
                ).reshape(_B, _NCAND)
    cand_vals = jnp.take_along_axis(scores, cand_idx, axis=1)

    pos = pl.pallas_call(
        _candpick_kernel,
        grid=(_B // _BT,),
        in_specs=[pl.BlockSpec((_BT, _NCAND), lambda i: (i, 0))],
        out_specs=pl.BlockSpec((_BT, LARGEST_RETRIEVAL), lambda i: (i, 0)),
        out_shape=jax.ShapeDtypeStruct((_B, LARGEST_RETRIEVAL), jnp.int32),
    )(cand_vals)
    I = jnp.take_along_axis(cand_idx, pos, axis=1)

    retrieved_labels = train_labels[I]
    is_neg = retrieved_labels != query_labels[:, None]
    is_pos = jnp.logical_not(is_neg)
    r_hn, f_hn = _sel_ranks(is_neg, NO_HARD_NEG)
    r_pp, f_pp = _sel_ranks(is_pos, NO_PSEUDO_POS)
    I_hn = jnp.take_along_axis(I, r_hn, axis=1)
    I_pp = jnp.take_along_axis(I, r_pp, axis=1)
    hard_negative_features = jnp.where(f_hn[..., None], t[I_hn], 0.0)
    pseudo_positive_features = jnp.where(f_pp[..., None], t[I_pp], 0.0)
    return hard_negative_features, pseudo_positive_features


# speedup vs baseline: 1.1188x; 1.1188x over previous
import functools

import jax
import jax.numpy as jnp
from jax.experimental import pallas as pl
from jax.experimental.pallas import tpu as pltpu

LARGEST_RETRIEVAL = 64
NO_HARD_NEG = 10
NO_PSEUDO_POS = 2

_B = 1024
_D = 128
_N = 100000
_BT = 256
_W = 4096
_NPAD = 102400
_BIN = 32
_NBINS_STEP = _W // _BIN
_MPAD = _NPAD // _BIN
_NCAND = LARGEST_RETRIEVAL * _BIN


def _l2n(x):
    n = jnp.linalg.norm(x, axis=-1, keepdims=True)
    return x / jnp.maximum(n, 1e-12)


def _score_binmax_kernel(q_ref, sq_ref, t_ref, st_ref, s_ref, m_ref):
    s = jax.lax.dot_general(
        q_ref[...], t_ref[...], (((1,), (1,)), ((), ())),
        preferred_element_type=jnp.float32,
    )
    neg_d2 = -((sq_ref[...] + st_ref[0]) - 2.0 * s)

    j = pl.program_id(1)
    gid = j * _W + jax.lax.broadcasted_iota(jnp.int32, (_BT, _W), 1)
    v = jnp.where(gid < _N, neg_d2, -1e30)
    s_ref[...] = v
    m_ref[...] = jnp.max(v.reshape(_BT, _NBINS_STEP, _BIN), axis=2)


def _topk_iter(v, k):
    rows, L = v.shape
    lane = jax.lax.broadcasted_iota(jnp.int32, (rows, L), 1)

    m = v
    cols = []
    for _ in range(k):
        mx = jnp.max(m, axis=1, keepdims=True)
        am = jnp.min(jnp.where(m == mx, lane, L), axis=1)
        cols.append(am[:, None])
        m = jnp.where(lane == am[:, None], -jnp.inf, m)
    return jnp.concatenate(cols, axis=1)


def _binpick_kernel(m_ref, ids_ref):
    ids = _topk_iter(m_ref[...], LARGEST_RETRIEVAL)
    cols = []
    for _ in range(LARGEST_RETRIEVAL):
        mn = jnp.min(ids, axis=1, keepdims=True)
        cols.append(mn)
        ids = jnp.where(ids == mn, _MPAD, ids)
    ids_ref[...] = jnp.concatenate(cols, axis=1)


def _candpick_kernel(cv_ref, pos_ref):
    pos_ref[...] = _topk_iter(cv_ref[...], LARGEST_RETRIEVAL)


def _sel_ranks(mask, quota):
    rank = jnp.cumsum(mask.astype(jnp.int32), axis=1) - 1
    valid = mask & (rank < quota)
    oh = valid[:, :, None] & (rank[:, :, None] == jnp.arange(quota)[None, None, :])
    r_s = jnp.sum(jnp.where(oh, jnp.arange(mask.shape[1])[None, :, None], 0),
                  axis=1)
    filled = jnp.any(oh, axis=1)
    return r_s, filled


@jax.jit
def kernel(query_feats, query_labels, train_feats, train_labels):
    q = _l2n(query_feats)
    t = _l2n(train_feats)
    sq = jnp.sum(q * q, axis=1)
    st = jnp.sum(t * t, axis=1)

    t_pad = jnp.pad(t, ((0, _NPAD - _N), (0, 0)))
    st_pad = jnp.pad(st, (0, _NPAD - _N)).reshape(_NPAD // _W, 1, _W)
    sq2 = sq.reshape(_B, 1)

    scores, binmax = pl.pallas_call(
        _score_binmax_kernel,
        grid=(_B // _BT, _NPAD // _W),
        in_specs=[
            pl.BlockSpec((_BT, _D), lambda i, j: (i, 0)),
            pl.BlockSpec((_BT, 1), lambda i, j: (i, 0)),
            pl.BlockSpec((_W, _D), lambda i, j: (j, 0)),
            pl.BlockSpec((1, 1, _W), lambda i, j: (j, 0, 0)),
        ],
        out_specs=[
            pl.BlockSpec((_BT, _W), lambda i, j: (i, j)),
            pl.BlockSpec((_BT, _NBINS_STEP), lambda i, j: (i, j)),
        ],
        out_shape=[
            jax.ShapeDtypeStruct((_B, _NPAD), jnp.float32),
            jax.ShapeDtypeStruct((_B, _MPAD), jnp.float32),
        ],
    )(q, sq2, t_pad, st_pad)

    bin_ids = pl.pallas_call(
        _binpick_kernel,
        grid=(_B // _BT,),
        in_specs=[pl.BlockSpec((_BT, _MPAD), lambda i: (i, 0))],
        out_specs=pl.BlockSpec((_BT, LARGEST_RETRIEVAL), lambda i: (i, 0)),
        out_shape=jax.ShapeDtypeStruct((_B, LARGEST_RETRIEVAL), jnp.int32),
    )(binmax)

    col_base = bin_ids * _BIN
    cand_idx = (col_base[:, :, None] + jnp.arange(_BIN)[None, None, :]
                ).reshape(_B, _NCAND)
    cand_vals = jnp.take_along_axis(scores, cand_idx, axis=1)

    pos = pl.pallas_call(
        _candpick_kernel,
        grid=(_B // _BT,),
        in_specs=[pl.BlockSpec((_BT, _NCAND), lambda i: (i, 0))],
        out_specs=pl.BlockSpec((_BT, LARGEST_RETRIEVAL), lambda i: (i, 0)),
        out_shape=jax.ShapeDtypeStruct((_B, LARGEST_RETRIEVAL), jnp.int32),
    )(cand_vals)
    I = jnp.take_along_axis(cand_idx, pos, axis=1)

    retrieved_labels = train_labels[I]
    is_neg = retrieved_labels != query_labels[:, None]
    is_pos = jnp.logical_not(is_neg)
    r_hn, f_hn = _sel_ranks(is_neg, NO_HARD_NEG)
    r_pp, f_pp = _sel_ranks(is_pos, NO_PSEUDO_POS)
    I_hn = jnp.take_along_axis(I, r_hn, axis=1)
    I_pp = jnp.take_along_axis(I, r_pp, axis=1)
    hard_negative_features = jnp.where(f_hn[..., None], t[I_hn], 0.0)
    pseudo_positive_features = jnp.where(f_pp[..., None], t[I_pp], 0.0)
    return hard_negative_features, pseudo_positive_features
